# baseline (speedup 1.0000x reference)
import functools

import jax
import jax.numpy as jnp
from jax import lax
from jax.experimental import pallas as pl
from jax.experimental.pallas import tpu as pltpu
from jax.experimental.pallas import tpu_sc as plsc

H = 1024
E = 64
EP = 128
R = 8
K = 8
SCALING = 16.0 / float(R)
GAMMA = 0.5
SOFT_T = 0.5
EPS = 1e-6

NC = 2
NS = 16
NW = NC * NS
CHUNK = 32
TBLK = 512


def _sc_gather(ids, emb_table, aflat):
    ntok = ids.shape[0]
    vocab = emb_table.shape[0]
    tok_per_w = ntok // NW
    nchunk = tok_per_w // CHUNK
    mesh = plsc.VectorSubcoreMesh(core_axis_name="c", subcore_axis_name="s")

    @functools.partial(
        pl.kernel,
        mesh=mesh,
        out_type=(
            jax.ShapeDtypeStruct((ntok, H), jnp.float32),
            jax.ShapeDtypeStruct((NW, R, tok_per_w), jnp.float32),
        ),
        scratch_types=[
            pltpu.VMEM((tok_per_w,), jnp.int32),
            pltpu.VMEM((R, tok_per_w), jnp.int32),
            pltpu.VMEM((R, tok_per_w), jnp.float32),
            pltpu.VMEM((CHUNK, H), jnp.float32),
            pltpu.VMEM((CHUNK, H), jnp.float32),
            pltpu.SemaphoreType.DMA,
            pltpu.SemaphoreType.DMA,
            pltpu.SemaphoreType.DMA,
        ],
    )
    def k(ids_hbm, emb_hbm, aflat_hbm, out_emb, out_a,
          idx_v, idxr_v, avals_v, rows0, rows1, sem_e0, sem_e1, sem_a):
        wid = lax.axis_index("s") * NC + lax.axis_index("c")
        base = wid * tok_per_w
        pltpu.sync_copy(ids_hbm.at[pl.ds(base, tok_per_w)], idx_v)
        for r in range(R):
            for j in range(tok_per_w // 16):
                sl = pl.ds(j * 16, 16)
                idxr_v[r, sl] = idx_v[sl] + r * vocab

        def start(g, rows, sem_e):
            isl = pl.ds(g * CHUNK, CHUNK)
            pltpu.async_copy(emb_hbm.at[idx_v.at[isl]], rows, sem_e)
            for r in range(R):
                pltpu.async_copy(aflat_hbm.at[idxr_v.at[r, isl]],
                                 avals_v.at[r, isl], sem_a)

        def drain(g, rows, sem_e):
            pltpu.make_async_copy(emb_hbm.at[idx_v.at[pl.ds(0, CHUNK)]],
                                  rows, sem_e).wait()
            isl = pl.ds(0, CHUNK)
            for r in range(R):
                pltpu.make_async_copy(aflat_hbm.at[idxr_v.at[r, isl]],
                                      avals_v.at[r, isl], sem_a).wait()
            off = pl.multiple_of(base + g * CHUNK, CHUNK)
            pltpu.sync_copy(rows, out_emb.at[pl.ds(off, CHUNK)])

        start(0, rows0, sem_e0)

        def body(go, carry):
            g0 = 2 * go
            start(g0 + 1, rows1, sem_e1)
            drain(g0, rows0, sem_e0)

            @pl.when(g0 + 2 < nchunk)
            def _():
                start(g0 + 2, rows0, sem_e0)

            drain(g0 + 1, rows1, sem_e1)
            return carry

        lax.fori_loop(0, nchunk // 2, body, 0)

        for r in range(R):
            pltpu.sync_copy(avals_v.at[r], out_a.at[wid, r])

    return k(ids, emb_table, aflat)


def _fused_body(nb, b128_ref, base_ref, a1_ref, a2_ref, lb_ref, lb64t_ref,
                limes_ref, eye_ref, out_ref, hacc, dacc):
    i = pl.program_id(0)

    @pl.when(i == 0)
    def _():
        hacc[...] = jnp.full(hacc.shape, EPS, jnp.float32)
        dacc[...] = jnp.full(dacc.shape, EPS / SCALING, jnp.float32)

    @pl.when(i < nb)
    def _():
        a8t = a1_ref[0]
        d64t = jnp.dot(lb64t_ref[...], a8t,
                       preferred_element_type=jnp.float32)
        le = lax.broadcasted_iota(jnp.int32, (TBLK, EP), 1)
        b64 = jnp.where(le < E, b128_ref[...], 0.0)
        hacc[...] = jnp.maximum(hacc[...], jnp.abs(b64))
        dacc[...] = jnp.maximum(dacc[...], jnp.abs(d64t))

    @pl.when(i >= nb)
    def _():
        a8t = a2_ref[0]
        delta = lax.dot_general(a8t, lb_ref[...], (((0,), (0,)), ((), ())),
                                preferred_element_type=jnp.float32)
        delta = delta * SCALING
        d64t = jnp.dot(lb64t_ref[...], a8t,
                       preferred_element_type=jnp.float32)
        base = base_ref[...]
        b64t = lax.dot_general(eye_ref[...], base[:, :EP],
                               (((1,), (1,)), ((), ())),
                               preferred_element_type=jnp.float32,
                               precision=lax.Precision.HIGHEST)
        h = jnp.max(hacc[...])
        d = SCALING * jnp.max(dacc[...])
        c1 = (1.0 - GAMMA) / h
        c2 = GAMMA * SCALING / d
        logits = b64t * c1 + d64t * c2
        lm = jnp.max(logits, axis=0, keepdims=True)
        u = jnp.exp(logits - lm)
        s = jnp.sum(u, axis=0, keepdims=True)
        r = 1.0 / s
        le = lax.broadcasted_iota(jnp.int32, (E, TBLK), 0)
        tmp = u
        for _ in range(K - 1):
            mx = jnp.max(tmp, axis=0, keepdims=True)
            ismx = tmp >= mx
            fidx = jnp.min(jnp.where(ismx, le, E), axis=0, keepdims=True)
            tmp = jnp.where(le == fidx, -1.0, tmp)
        uthr = jnp.max(tmp, axis=0, keepdims=True)
        msk = jax.nn.sigmoid((u - uthr) * ((1.0 / SOFT_T) * r))
        um = u * msk
        w = um * (1.0 / (jnp.sum(um, axis=0, keepdims=True) + 1e-9 * s))
        p_mix = lax.dot_general(w, limes_ref[...], (((0,), (0,)), ((), ())),
                                preferred_element_type=jnp.float32)
        out_ref[...] = base + delta * p_mix


def _tc_pipeline(base_buf, a8, lora_B, lb64t, limes):
    ntok = base_buf.shape[0]
    nb = ntok // TBLK
    tok_per_w = ntok // NW
    bpw = tok_per_w // TBLK
    body = functools.partial(_fused_body, nb)

    def _a_map(j):
        return (j // bpw, 0, j % bpw)

    out = pl.pallas_call(
        body,
        grid=(2 * nb,),
        in_specs=[
            pl.BlockSpec((TBLK, EP),
                         lambda i: (jnp.minimum(i, nb - 1), 0)),
            pl.BlockSpec((TBLK, H),
                         lambda i: (jnp.maximum(i - nb, 0), 0)),
            pl.BlockSpec((1, R, TBLK),
                         lambda i: _a_map(jnp.minimum(i, nb - 1))),
            pl.BlockSpec((1, R, TBLK),
                         lambda i: _a_map(jnp.maximum(i - nb, 0))),
            pl.BlockSpec((R, H), lambda i: (0, 0)),
            pl.BlockSpec((E, R), lambda i: (0, 0)),
            pl.BlockSpec((E, H), lambda i: (0, 0)),
            pl.BlockSpec((E, EP), lambda i: (0, 0)),
        ],
        out_specs=pl.BlockSpec((TBLK, H), lambda i: (jnp.maximum(i - nb, 0), 0)),
        out_shape=jax.ShapeDtypeStruct((ntok, H), jnp.float32),
        scratch_shapes=[
            pltpu.VMEM((TBLK, EP), jnp.float32),
            pltpu.VMEM((E, TBLK), jnp.float32),
        ],
        compiler_params=pltpu.CompilerParams(
            dimension_semantics=("arbitrary",)),
    )(base_buf, base_buf, a8, a8, lora_B, lb64t, limes,
      jnp.eye(E, EP, dtype=jnp.float32))
    return out


def kernel(input_ids, emb_table, lora_A, lora_B, limes):
    bsz, t = input_ids.shape
    ids = input_ids.reshape(-1).astype(jnp.int32)
    aflat = jnp.transpose(lora_A).reshape(-1)
    base_buf, a8 = _sc_gather(ids, emb_table, aflat)
    lb64t = jnp.transpose(lora_B[:, :E])
    out = _tc_pipeline(base_buf, a8, lora_B, lb64t, limes)
    return out.reshape(bsz, t, H)


# speedup vs baseline: 1.0251x; 1.0251x over previous
import functools

import jax
import jax.numpy as jnp
from jax import lax
from jax.experimental import pallas as pl
from jax.experimental.pallas import tpu as pltpu
from jax.experimental.pallas import tpu_sc as plsc

H = 1024
E = 64
EP = 128
R = 8
K = 8
SCALING = 16.0 / float(R)
GAMMA = 0.5
SOFT_T = 0.5
EPS = 1e-6

NC = 2
NS = 16
NW = NC * NS
CHUNK = 32
TBLK = 512


def _sc_gather(ids, emb_table, aflat):
    ntok = ids.shape[0]
    vocab = emb_table.shape[0]
    tok_per_w = ntok // NW
    nchunk = tok_per_w // CHUNK
    mesh = plsc.VectorSubcoreMesh(core_axis_name="c", subcore_axis_name="s")

    @functools.partial(
        pl.kernel,
        mesh=mesh,
        out_type=(
            jax.ShapeDtypeStruct((ntok, H), jnp.float32),
            jax.ShapeDtypeStruct((NW, R, tok_per_w), jnp.float32),
        ),
        scratch_types=[
            pltpu.VMEM((tok_per_w,), jnp.int32),
            pltpu.VMEM((R, tok_per_w), jnp.int32),
            pltpu.VMEM((R, tok_per_w), jnp.float32),
            pltpu.VMEM((CHUNK, H), jnp.float32),
            pltpu.VMEM((CHUNK, H), jnp.float32),
            pltpu.SemaphoreType.DMA,
            pltpu.SemaphoreType.DMA,
            pltpu.SemaphoreType.DMA,
        ],
    )
    def k(ids_hbm, emb_hbm, aflat_hbm, out_emb, out_a,
          idx_v, idxr_v, avals_v, rows0, rows1, sem_e0, sem_e1, sem_a):
        wid = lax.axis_index("s") * NC + lax.axis_index("c")
        base = wid * tok_per_w
        pltpu.sync_copy(ids_hbm.at[pl.ds(base, tok_per_w)], idx_v)
        for r in range(R):
            for j in range(tok_per_w // 16):
                sl = pl.ds(j * 16, 16)
                idxr_v[r, sl] = idx_v[sl] + r * vocab

        def start(g, rows, sem_e):
            isl = pl.ds(g * CHUNK, CHUNK)
            pltpu.async_copy(emb_hbm.at[idx_v.at[isl]], rows, sem_e)
            for r in range(R):
                pltpu.async_copy(aflat_hbm.at[idxr_v.at[r, isl]],
                                 avals_v.at[r, isl], sem_a)

        def drain(g, rows, sem_e):
            pltpu.make_async_copy(emb_hbm.at[idx_v.at[pl.ds(0, CHUNK)]],
                                  rows, sem_e).wait()
            isl = pl.ds(0, CHUNK)
            for r in range(R):
                pltpu.make_async_copy(aflat_hbm.at[idxr_v.at[r, isl]],
                                      avals_v.at[r, isl], sem_a).wait()
            off = pl.multiple_of(base + g * CHUNK, CHUNK)
            pltpu.sync_copy(rows, out_emb.at[pl.ds(off, CHUNK)])

        start(0, rows0, sem_e0)

        def body(go, carry):
            g0 = 2 * go
            start(g0 + 1, rows1, sem_e1)
            drain(g0, rows0, sem_e0)

            @pl.when(g0 + 2 < nchunk)
            def _():
                start(g0 + 2, rows0, sem_e0)

            drain(g0 + 1, rows1, sem_e1)
            return carry

        lax.fori_loop(0, nchunk // 2, body, 0)

        for r in range(R):
            pltpu.sync_copy(avals_v.at[r], out_a.at[wid, r])

    return k(ids, emb_table, aflat)


def _fused_body(nb, b128_ref, base_ref, a1_ref, a2_ref, lb_ref, lb64t_ref,
                limes_ref, eye_ref, out_ref, hacc, dacc):
    i = pl.program_id(0)

    @pl.when(i == 0)
    def _():
        hacc[...] = jnp.full(hacc.shape, EPS, jnp.float32)
        dacc[...] = jnp.full(dacc.shape, EPS / SCALING, jnp.float32)

    @pl.when(i < nb)
    def _():
        a8t = a1_ref[0]
        d64t = jnp.dot(lb64t_ref[...], a8t,
                       preferred_element_type=jnp.float32)
        le = lax.broadcasted_iota(jnp.int32, (TBLK, EP), 1)
        b64 = jnp.where(le < E, b128_ref[...], 0.0)
        hacc[...] = jnp.maximum(hacc[...], jnp.abs(b64))
        dacc[...] = jnp.maximum(dacc[...], jnp.abs(d64t))

    @pl.when(i >= nb)
    def _():
        a8t = a2_ref[0]
        delta = lax.dot_general(a8t, lb_ref[...], (((0,), (0,)), ((), ())),
                                preferred_element_type=jnp.float32)
        delta = delta * SCALING
        d64t = jnp.dot(lb64t_ref[...], a8t,
                       preferred_element_type=jnp.float32)
        base = base_ref[...]
        b64t = lax.dot_general(eye_ref[...], base[:, :EP],
                               (((1,), (1,)), ((), ())),
                               preferred_element_type=jnp.float32)
        h = jnp.max(hacc[...])
        d = SCALING * jnp.max(dacc[...])
        c1 = (1.0 - GAMMA) / h
        c2 = GAMMA * SCALING / d
        logits = b64t * c1 + d64t * c2
        lm = jnp.max(logits, axis=0, keepdims=True)
        u = jnp.exp(logits - lm)
        s = jnp.sum(u, axis=0, keepdims=True)
        r = 1.0 / s
        le = lax.broadcasted_iota(jnp.int32, (E, TBLK), 0)
        tmp = u
        for _ in range(K - 1):
            mx = jnp.max(tmp, axis=0, keepdims=True)
            ismx = tmp >= mx
            fidx = jnp.min(jnp.where(ismx, le, E), axis=0, keepdims=True)
            tmp = jnp.where(le == fidx, -1.0, tmp)
        uthr = jnp.max(tmp, axis=0, keepdims=True)
        msk = jax.nn.sigmoid((u - uthr) * ((1.0 / SOFT_T) * r))
        um = u * msk
        w = um * (1.0 / (jnp.sum(um, axis=0, keepdims=True) + 1e-9 * s))
        p_mix = lax.dot_general(w, limes_ref[...], (((0,), (0,)), ((), ())),
                                preferred_element_type=jnp.float32)
        out_ref[...] = base + delta * p_mix


def _tc_pipeline(base_buf, a8, lora_B, lb64t, limes):
    ntok = base_buf.shape[0]
    nb = ntok // TBLK
    tok_per_w = ntok // NW
    bpw = tok_per_w // TBLK
    body = functools.partial(_fused_body, nb)

    def _a_map(j):
        return (j // bpw, 0, j % bpw)

    out = pl.pallas_call(
        body,
        grid=(2 * nb,),
        in_specs=[
            pl.BlockSpec((TBLK, EP),
                         lambda i: (jnp.minimum(i, nb - 1), 0)),
            pl.BlockSpec((TBLK, H),
                         lambda i: (jnp.maximum(i - nb, 0), 0)),
            pl.BlockSpec((1, R, TBLK),
                         lambda i: _a_map(jnp.minimum(i, nb - 1))),
            pl.BlockSpec((1, R, TBLK),
                         lambda i: _a_map(jnp.maximum(i - nb, 0))),
            pl.BlockSpec((R, H), lambda i: (0, 0)),
            pl.BlockSpec((E, R), lambda i: (0, 0)),
            pl.BlockSpec((E, H), lambda i: (0, 0)),
            pl.BlockSpec((E, EP), lambda i: (0, 0)),
        ],
        out_specs=pl.BlockSpec((TBLK, H), lambda i: (jnp.maximum(i - nb, 0), 0)),
        out_shape=jax.ShapeDtypeStruct((ntok, H), jnp.float32),
        scratch_shapes=[
            pltpu.VMEM((TBLK, EP), jnp.float32),
            pltpu.VMEM((E, TBLK), jnp.float32),
        ],
        compiler_params=pltpu.CompilerParams(
            dimension_semantics=("arbitrary",)),
    )(base_buf, base_buf, a8, a8, lora_B, lb64t, limes,
      jnp.eye(E, EP, dtype=jnp.float32))
    return out


def kernel(input_ids, emb_table, lora_A, lora_B, limes):
    bsz, t = input_ids.shape
    ids = input_ids.reshape(-1).astype(jnp.int32)
    aflat = jnp.transpose(lora_A).reshape(-1)
    base_buf, a8 = _sc_gather(ids, emb_table, aflat)
    lb64t = jnp.transpose(lora_B[:, :E])
    out = _tc_pipeline(base_buf, a8, lora_B, lb64t, limes)
    return out.reshape(bsz, t, H)


# speedup vs baseline: 1.1829x; 1.1540x over previous
import functools

import jax
import jax.numpy as jnp
from jax import lax
from jax.experimental import pallas as pl
from jax.experimental.pallas import tpu as pltpu
from jax.experimental.pallas import tpu_sc as plsc

H = 1024
E = 64
EP = 128
R = 8
K = 8
SCALING = 16.0 / float(R)
GAMMA = 0.5
SOFT_T = 0.5
EPS = 1e-6

NC = 2
NS = 16
NW = NC * NS
CHUNK = 32
TBLK = 512


def _sc_gather(ids, emb_table, aflat):
    ntok = ids.shape[0]
    vocab = emb_table.shape[0]
    tok_per_w = ntok // NW
    nchunk = tok_per_w // CHUNK
    mesh = plsc.VectorSubcoreMesh(core_axis_name="c", subcore_axis_name="s")

    @functools.partial(
        pl.kernel,
        mesh=mesh,
        out_type=(
            jax.ShapeDtypeStruct((ntok, H), jnp.float32),
            jax.ShapeDtypeStruct((NW, R, tok_per_w), jnp.float32),
            jax.ShapeDtypeStruct((NW, 16), jnp.float32),
        ),
        scratch_types=[
            pltpu.VMEM((tok_per_w,), jnp.int32),
            pltpu.VMEM((R, tok_per_w), jnp.int32),
            pltpu.VMEM((R, tok_per_w), jnp.float32),
            pltpu.VMEM((16,), jnp.float32),
            pltpu.VMEM((CHUNK, H), jnp.float32),
            pltpu.VMEM((CHUNK, H), jnp.float32),
            pltpu.SemaphoreType.DMA,
            pltpu.SemaphoreType.DMA,
            pltpu.SemaphoreType.DMA,
        ],
    )
    def k(ids_hbm, emb_hbm, aflat_hbm, out_emb, out_a, out_h,
          idx_v, idxr_v, avals_v, hacc_v, rows0, rows1,
          sem_e0, sem_e1, sem_a):
        wid = lax.axis_index("s") * NC + lax.axis_index("c")
        base = wid * tok_per_w
        pltpu.sync_copy(ids_hbm.at[pl.ds(base, tok_per_w)], idx_v)
        for r in range(R):
            for j in range(tok_per_w // 16):
                sl = pl.ds(j * 16, 16)
                idxr_v[r, sl] = idx_v[sl] + r * vocab
        hacc_v[...] = jnp.zeros((16,), jnp.float32)

        def start(g, rows, sem_e):
            isl = pl.ds(g * CHUNK, CHUNK)
            pltpu.async_copy(emb_hbm.at[idx_v.at[isl]], rows, sem_e)
            for r in range(R):
                pltpu.async_copy(aflat_hbm.at[idxr_v.at[r, isl]],
                                 avals_v.at[r, isl], sem_a)

        def drain(g, rows, sem_e):
            pltpu.make_async_copy(emb_hbm.at[idx_v.at[pl.ds(0, CHUNK)]],
                                  rows, sem_e).wait()
            isl = pl.ds(0, CHUNK)
            for r in range(R):
                pltpu.make_async_copy(aflat_hbm.at[idxr_v.at[r, isl]],
                                      avals_v.at[r, isl], sem_a).wait()
            off = pl.multiple_of(base + g * CHUNK, CHUNK)
            pltpu.sync_copy(rows, out_emb.at[pl.ds(off, CHUNK)])
            acc = hacc_v[...]
            for t in range(CHUNK):
                for j in range(E // 16):
                    acc = jnp.maximum(acc, jnp.abs(rows[t, pl.ds(j * 16, 16)]))
            hacc_v[...] = acc

        start(0, rows0, sem_e0)

        def body(go, carry):
            g0 = 2 * go
            start(g0 + 1, rows1, sem_e1)
            drain(g0, rows0, sem_e0)

            @pl.when(g0 + 2 < nchunk)
            def _():
                start(g0 + 2, rows0, sem_e0)

            drain(g0 + 1, rows1, sem_e1)
            return carry

        lax.fori_loop(0, nchunk // 2, body, 0)

        for r in range(R):
            pltpu.sync_copy(avals_v.at[r], out_a.at[wid, r])
        pltpu.sync_copy(hacc_v, out_h.at[wid])

    return k(ids, emb_table, aflat)


def _fused_body(nb, base_ref, a8_ref, a8full_ref, hpart_ref, lb_ref,
                lb64t_ref, limes_ref, out_ref, hd_s):
    i = pl.program_id(0)

    @pl.when(i == 0)
    def _():
        h = jnp.maximum(jnp.max(hpart_ref[...]), EPS)
        dm = jnp.float32(0.0)
        for w in range(NW):
            d64t_w = jnp.dot(lb64t_ref[...], a8full_ref[w],
                             preferred_element_type=jnp.float32)
            dm = jnp.maximum(dm, jnp.max(jnp.abs(d64t_w)))
        d = jnp.maximum(SCALING * dm, EPS)
        hd_s[0] = (1.0 - GAMMA) / h
        hd_s[1] = GAMMA * SCALING / d

    @pl.when(i > 0)
    def _():
        a8t = a8_ref[0]
        delta = lax.dot_general(a8t, lb_ref[...], (((0,), (0,)), ((), ())),
                                preferred_element_type=jnp.float32)
        delta = delta * SCALING
        d64t = jnp.dot(lb64t_ref[...], a8t,
                       preferred_element_type=jnp.float32)
        base = base_ref[...]
        b64t = jnp.transpose(base[:, :EP])[:E, :]
        logits = b64t * hd_s[0] + d64t * hd_s[1]
        lm = jnp.max(logits, axis=0, keepdims=True)
        u = jnp.exp(logits - lm)
        s = jnp.sum(u, axis=0, keepdims=True)
        r = 1.0 / s
        le = lax.broadcasted_iota(jnp.int32, (E, TBLK), 0)
        tmp = u
        for _ in range(K - 1):
            mx = jnp.max(tmp, axis=0, keepdims=True)
            ismx = tmp >= mx
            fidx = jnp.min(jnp.where(ismx, le, E), axis=0, keepdims=True)
            tmp = jnp.where(le == fidx, -1.0, tmp)
        uthr = jnp.max(tmp, axis=0, keepdims=True)
        msk = jax.nn.sigmoid((u - uthr) * ((1.0 / SOFT_T) * r))
        um = u * msk
        w = um * (1.0 / (jnp.sum(um, axis=0, keepdims=True) + 1e-9 * s))
        p_mix = lax.dot_general(w, limes_ref[...], (((0,), (0,)), ((), ())),
                                preferred_element_type=jnp.float32)
        out_ref[...] = base + delta * p_mix


def _tc_pipeline(base_buf, a8, hpart, lora_B, lb64t, limes):
    ntok = base_buf.shape[0]
    nb = ntok // TBLK
    tok_per_w = ntok // NW
    bpw = tok_per_w // TBLK
    body = functools.partial(_fused_body, nb)

    def _a_map(j):
        return (j // bpw, 0, j % bpw)

    out = pl.pallas_call(
        body,
        grid=(nb + 1,),
        in_specs=[
            pl.BlockSpec((TBLK, H),
                         lambda i: (jnp.maximum(i - 1, 0), 0)),
            pl.BlockSpec((1, R, TBLK),
                         lambda i: _a_map(jnp.maximum(i - 1, 0))),
            pl.BlockSpec((NW, R, tok_per_w), lambda i: (0, 0, 0)),
            pl.BlockSpec((NW, 16), lambda i: (0, 0)),
            pl.BlockSpec((R, H), lambda i: (0, 0)),
            pl.BlockSpec((E, R), lambda i: (0, 0)),
            pl.BlockSpec((E, H), lambda i: (0, 0)),
        ],
        out_specs=pl.BlockSpec((TBLK, H), lambda i: (jnp.maximum(i - 1, 0), 0)),
        out_shape=jax.ShapeDtypeStruct((ntok, H), jnp.float32),
        scratch_shapes=[
            pltpu.SMEM((2,), jnp.float32),
        ],
        compiler_params=pltpu.CompilerParams(
            dimension_semantics=("arbitrary",)),
    )(base_buf, a8, a8, hpart, lora_B, lb64t, limes)
    return out


def kernel(input_ids, emb_table, lora_A, lora_B, limes):
    bsz, t = input_ids.shape
    ids = input_ids.reshape(-1).astype(jnp.int32)
    aflat = jnp.transpose(lora_A).reshape(-1)
    base_buf, a8, hpart = _sc_gather(ids, emb_table, aflat)
    lb64t = jnp.transpose(lora_B[:, :E])
    out = _tc_pipeline(base_buf, a8, hpart, lora_B, lb64t, limes)
    return out.reshape(bsz, t, H)


# speedup vs baseline: 1.3103x; 1.1076x over previous
import functools

import jax
import jax.numpy as jnp
from jax import lax
from jax.experimental import pallas as pl
from jax.experimental.pallas import tpu as pltpu
from jax.experimental.pallas import tpu_sc as plsc

H = 1024
E = 64
EP = 128
R = 8
K = 8
SCALING = 16.0 / float(R)
GAMMA = 0.5
SOFT_T = 0.5
EPS = 1e-6

NC = 2
NS = 16
NW = NC * NS
CHUNK = 32
TBLK = 1024


def _sc_gather(ids, emb_table, aflat):
    ntok = ids.shape[0]
    vocab = emb_table.shape[0]
    tok_per_w = ntok // NW
    nchunk = tok_per_w // CHUNK
    mesh = plsc.VectorSubcoreMesh(core_axis_name="c", subcore_axis_name="s")

    @functools.partial(
        pl.kernel,
        mesh=mesh,
        out_type=(
            jax.ShapeDtypeStruct((ntok, H), jnp.float32),
            jax.ShapeDtypeStruct((NW, R, tok_per_w), jnp.float32),
            jax.ShapeDtypeStruct((NW, 16), jnp.float32),
        ),
        scratch_types=[
            pltpu.VMEM((tok_per_w,), jnp.int32),
            pltpu.VMEM((R, tok_per_w), jnp.int32),
            pltpu.VMEM((R, tok_per_w), jnp.float32),
            pltpu.VMEM((16,), jnp.float32),
            pltpu.VMEM((CHUNK, H), jnp.float32),
            pltpu.VMEM((CHUNK, H), jnp.float32),
            pltpu.SemaphoreType.DMA,
            pltpu.SemaphoreType.DMA,
            pltpu.SemaphoreType.DMA,
        ],
    )
    def k(ids_hbm, emb_hbm, aflat_hbm, out_emb, out_a, out_h,
          idx_v, idxr_v, avals_v, hacc_v, rows0, rows1,
          sem_e0, sem_e1, sem_a):
        wid = lax.axis_index("s") * NC + lax.axis_index("c")
        base = wid * tok_per_w
        pltpu.sync_copy(ids_hbm.at[pl.ds(base, tok_per_w)], idx_v)
        for r in range(R):
            for j in range(tok_per_w // 16):
                sl = pl.ds(j * 16, 16)
                idxr_v[r, sl] = idx_v[sl] + r * vocab
        hacc_v[...] = jnp.zeros((16,), jnp.float32)

        def start(g, rows, sem_e):
            isl = pl.ds(g * CHUNK, CHUNK)
            pltpu.async_copy(emb_hbm.at[idx_v.at[isl]], rows, sem_e)
            for r in range(R):
                pltpu.async_copy(aflat_hbm.at[idxr_v.at[r, isl]],
                                 avals_v.at[r, isl], sem_a)

        def drain(g, rows, sem_e):
            pltpu.make_async_copy(emb_hbm.at[idx_v.at[pl.ds(0, CHUNK)]],
                                  rows, sem_e).wait()
            isl = pl.ds(0, CHUNK)
            for r in range(R):
                pltpu.make_async_copy(aflat_hbm.at[idxr_v.at[r, isl]],
                                      avals_v.at[r, isl], sem_a).wait()
            off = pl.multiple_of(base + g * CHUNK, CHUNK)
            pltpu.sync_copy(rows, out_emb.at[pl.ds(off, CHUNK)])
            acc = hacc_v[...]
            for t in range(CHUNK):
                for j in range(E // 16):
                    acc = jnp.maximum(acc, jnp.abs(rows[t, pl.ds(j * 16, 16)]))
            hacc_v[...] = acc

        start(0, rows0, sem_e0)

        def body(go, carry):
            g0 = 2 * go
            start(g0 + 1, rows1, sem_e1)
            drain(g0, rows0, sem_e0)

            @pl.when(g0 + 2 < nchunk)
            def _():
                start(g0 + 2, rows0, sem_e0)

            drain(g0 + 1, rows1, sem_e1)
            return carry

        lax.fori_loop(0, nchunk // 2, body, 0)

        for r in range(R):
            pltpu.sync_copy(avals_v.at[r], out_a.at[wid, r])
        pltpu.sync_copy(hacc_v, out_h.at[wid])

    return k(ids, emb_table, aflat)


def _fused_body(nb, base_ref, a8_ref, a8full_ref, hpart_ref, lb_ref,
                lb64t_ref, limes_ref, out_ref, hd_s):
    i = pl.program_id(0)

    @pl.when(i == 0)
    def _():
        h = jnp.maximum(jnp.max(hpart_ref[...]), EPS)
        dm = jnp.float32(0.0)
        for w in range(NW):
            d64t_w = jnp.dot(lb64t_ref[...], a8full_ref[w],
                             preferred_element_type=jnp.float32)
            dm = jnp.maximum(dm, jnp.max(jnp.abs(d64t_w)))
        d = jnp.maximum(SCALING * dm, EPS)
        hd_s[0] = (1.0 - GAMMA) / h
        hd_s[1] = GAMMA * SCALING / d

    @pl.when(i > 0)
    def _():
        a8t = a8_ref[0]
        delta = lax.dot_general(a8t, lb_ref[...], (((0,), (0,)), ((), ())),
                                preferred_element_type=jnp.float32)
        delta = delta * SCALING
        d64t = jnp.dot(lb64t_ref[...], a8t,
                       preferred_element_type=jnp.float32)
        base = base_ref[...]
        b64t = jnp.transpose(base[:, :EP])[:E, :]
        logits = b64t * hd_s[0] + d64t * hd_s[1]
        lm = jnp.max(logits, axis=0, keepdims=True)
        u = jnp.exp(logits - lm)
        s = jnp.sum(u, axis=0, keepdims=True)
        r = 1.0 / s
        le = lax.broadcasted_iota(jnp.int32, (E, TBLK), 0)
        tmp = u
        for _ in range(K - 1):
            mx = jnp.max(tmp, axis=0, keepdims=True)
            ismx = tmp >= mx
            fidx = jnp.min(jnp.where(ismx, le, E), axis=0, keepdims=True)
            tmp = jnp.where(le == fidx, -1.0, tmp)
        uthr = jnp.max(tmp, axis=0, keepdims=True)
        msk = jax.nn.sigmoid((u - uthr) * ((1.0 / SOFT_T) * r))
        um = u * msk
        w = um * (1.0 / (jnp.sum(um, axis=0, keepdims=True) + 1e-9 * s))
        p_mix = lax.dot_general(w, limes_ref[...], (((0,), (0,)), ((), ())),
                                preferred_element_type=jnp.float32)
        out_ref[...] = base + delta * p_mix


def _tc_pipeline(base_buf, a8, hpart, lora_B, lb64t, limes):
    ntok = base_buf.shape[0]
    nb = ntok // TBLK
    tok_per_w = ntok // NW
    bpw = tok_per_w // TBLK
    body = functools.partial(_fused_body, nb)

    def _a_map(j):
        return (j // bpw, 0, j % bpw)

    out = pl.pallas_call(
        body,
        grid=(nb + 1,),
        in_specs=[
            pl.BlockSpec((TBLK, H),
                         lambda i: (jnp.maximum(i - 1, 0), 0)),
            pl.BlockSpec((1, R, TBLK),
                         lambda i: _a_map(jnp.maximum(i - 1, 0))),
            pl.BlockSpec((NW, R, tok_per_w), lambda i: (0, 0, 0)),
            pl.BlockSpec((NW, 16), lambda i: (0, 0)),
            pl.BlockSpec((R, H), lambda i: (0, 0)),
            pl.BlockSpec((E, R), lambda i: (0, 0)),
            pl.BlockSpec((E, H), lambda i: (0, 0)),
        ],
        out_specs=pl.BlockSpec((TBLK, H), lambda i: (jnp.maximum(i - 1, 0), 0)),
        out_shape=jax.ShapeDtypeStruct((ntok, H), jnp.float32),
        scratch_shapes=[
            pltpu.SMEM((2,), jnp.float32),
        ],
        compiler_params=pltpu.CompilerParams(
            dimension_semantics=("arbitrary",)),
    )(base_buf, a8, a8, hpart, lora_B, lb64t, limes)
    return out


def kernel(input_ids, emb_table, lora_A, lora_B, limes):
    bsz, t = input_ids.shape
    ids = input_ids.reshape(-1).astype(jnp.int32)
    aflat = jnp.transpose(lora_A).reshape(-1)
    base_buf, a8, hpart = _sc_gather(ids, emb_table, aflat)
    lb64t = jnp.transpose(lora_B[:, :E])
    out = _tc_pipeline(base_buf, a8, hpart, lora_B, lb64t, limes)
    return out.reshape(bsz, t, H)
